# baseline (speedup 1.0000x reference)
import jax
import jax.numpy as jnp
from jax import lax
from jax.experimental import pallas as pl
from jax.experimental.pallas import tpu as pltpu
from jax.experimental.pallas import tpu_sc as plsc

M_PAD = 10240
K = 16
N_SP = 2500
N_PAD = 2560
C_IN = 256
C_PE = 32
C_MID = 16
C_OUT = 256
ROWS = M_PAD * K
TM = 256
TROWS = TM * K
GRID = M_PAD // TM

_NSLICE = 2
SM = M_PAD // _NSLICE
SROWS = SM * K
SGRID = SM // TM

_NW = 32
_PER_W = SROWS // _NW
_PTS_W = SM // _NW
_CH = 128
_NSUP = _PER_W // _CH


def _sc_body(feat_tbl, xyz_tbl, dxyz_flat, idx_hbm,
             out_feat, out_lx,
             idx_all, f_a, f_b, xg_a, xg_b, dxt,
             sf_a, sf_b, sg_a, sg_b, sw_a, sw_b, sx_a, sx_b):
    f32 = jnp.float32
    wid = lax.axis_index("s") * 2 + lax.axis_index("c")
    rbase = pl.multiple_of(wid * _PER_W, _PER_W)

    pltpu.sync_copy(idx_hbm.at[pl.ds(rbase, _PER_W)], idx_all)
    pltpu.sync_copy(dxyz_flat.at[pl.ds(pl.multiple_of(wid * _PTS_W * 16,
                                                      _PTS_W * 16),
                                       _PTS_W * 16)], dxt)

    def issue_gathers(su, fbuf, xbuf, fsem, xsem):
        sl = idx_all.at[pl.ds(su * _CH, _CH)]
        hf = pltpu.async_copy(feat_tbl.at[sl], fbuf, fsem)
        hx = pltpu.async_copy(xyz_tbl.at[sl], xbuf, xsem)
        return hf, hx

    def lx_compute(su, xbuf):
        def rows8(r2, carry):
            for u in range(8):
                r = r2 * 8 + u
                p = su * 8 + (r >> 4)
                v = xbuf[r, pl.ds(0, 16)]
                d = dxt[pl.ds(p * 16, 16)]
                xbuf[r, pl.ds(0, 16)] = v - d
            return carry

        lax.fori_loop(0, _CH // 8, rows8, 0)

    def super_out(su, fbuf, xbuf, fsem, xsem):
        off = pl.multiple_of(rbase + su * _CH, _CH)
        wf = pltpu.async_copy(fbuf, out_feat.at[pl.ds(off, _CH)], fsem)
        wx = pltpu.async_copy(xbuf, out_lx.at[pl.ds(off, _CH)], xsem)
        return wf, wx

    def body(j, carry):
        su0 = 2 * j
        su1 = 2 * j + 1
        gf_a, gx_a = issue_gathers(su0, f_a, xg_a, sf_a, sg_a)
        gf_b, gx_b = issue_gathers(su1, f_b, xg_b, sf_b, sg_b)

        gf_a.wait()
        gx_a.wait()
        wf_a = pltpu.async_copy(
            f_a, out_feat.at[pl.ds(pl.multiple_of(rbase + su0 * _CH, _CH), _CH)],
            sw_a)
        lx_compute(su0, xg_a)
        wx_a = pltpu.async_copy(
            xg_a, out_lx.at[pl.ds(pl.multiple_of(rbase + su0 * _CH, _CH), _CH)],
            sx_a)

        gf_b.wait()
        gx_b.wait()
        wf_b = pltpu.async_copy(
            f_b, out_feat.at[pl.ds(pl.multiple_of(rbase + su1 * _CH, _CH), _CH)],
            sw_b)
        lx_compute(su1, xg_b)
        wx_b = pltpu.async_copy(
            xg_b, out_lx.at[pl.ds(pl.multiple_of(rbase + su1 * _CH, _CH), _CH)],
            sx_b)

        wf_a.wait()
        wx_a.wait()
        wf_b.wait()
        wx_b.wait()
        return carry

    lax.fori_loop(0, _NSUP // 2, body, 0)


def _sc_gather(feat_tbl, xyz_tbl, dxyz_flat, idx):
    f32 = jnp.float32
    sc_call = pl.kernel(
        _sc_body,
        out_type=[
            jax.ShapeDtypeStruct((SROWS, C_IN // 2), jnp.int32),
            jax.ShapeDtypeStruct((SROWS, 128), f32),
        ],
        mesh=plsc.VectorSubcoreMesh(core_axis_name="c", subcore_axis_name="s",
                                    num_cores=2),
        scratch_types=[
            pltpu.VMEM((_PER_W,), jnp.int32),
            pltpu.VMEM((_CH, C_IN // 2), jnp.int32),
            pltpu.VMEM((_CH, C_IN // 2), jnp.int32),
            pltpu.VMEM((_CH, 128), f32),
            pltpu.VMEM((_CH, 128), f32),
            pltpu.VMEM((_PTS_W * 16,), f32),
            pltpu.SemaphoreType.DMA,
            pltpu.SemaphoreType.DMA,
            pltpu.SemaphoreType.DMA,
            pltpu.SemaphoreType.DMA,
            pltpu.SemaphoreType.DMA,
            pltpu.SemaphoreType.DMA,
            pltpu.SemaphoreType.DMA,
            pltpu.SemaphoreType.DMA,
        ],
    )
    return sc_call(feat_tbl, xyz_tbl, dxyz_flat, idx)


def _tc_body(gf_ref, lx_ref,  df_ref,
             wpe1_ref, bpe1_ref, wpe2_ref, bpe2_ref,
             wn1_ref, bn1_ref, wn2_ref, bn2_ref, wn3_ref, bn3_ref,
             wps_ref, wpp_ref, lb_ref,
             out_ref):
    f32 = jnp.float32
    lx8 = lx_ref[...][:, :8]

    def dot(a, b):
        return jnp.dot(a, b, preferred_element_type=f32)

    h = jnp.maximum(dot(lx8, wpe1_ref[...]) + bpe1_ref[...], 0.0)
    fpe = jnp.maximum(dot(h, wpe2_ref[...]) + bpe2_ref[...], 0.0)
    w1 = jnp.maximum(dot(lx8, wn1_ref[...]) + bn1_ref[...], 0.0)
    w2 = jnp.maximum(dot(w1, wn2_ref[...]) + bn2_ref[...], 0.0)
    wgt = jnp.maximum(dot(w2, wn3_ref[...]) + bn3_ref[...], 0.0)

    wgtT = wgt.T
    NP = 16
    NG = TM // NP
    GR = NP * K
    col = lax.broadcasted_iota(jnp.int32, (NP, GR), 1)
    rowp = lax.broadcasted_iota(jnp.int32, (NP, GR), 0)
    maskp = (col // K == rowp).astype(f32)

    x = gf_ref[...]
    gf_lo = lax.bitcast_convert_type(x << 16, f32)
    gf_hi = lax.bitcast_convert_type(x & jnp.int32(-65536), f32)
    gf2 = jnp.concatenate([gf_lo, gf_hi], axis=1)
    r2s = []
    r2p = []
    for g in range(NG):
        o3 = (wgtT[:, None, g * GR:(g + 1) * GR] * maskp[None, :, :])
        o3r = o3.reshape(GR, GR)
        r2s.append(dot(o3r, gf2[g * GR:(g + 1) * GR, :]))
        r2p.append(dot(o3r, fpe[g * GR:(g + 1) * GR, :]))

    acc = jnp.zeros((TM, C_OUT), f32)
    for mid in range(C_MID):
        gm_s = jnp.concatenate(
            [r[mid * NP:(mid + 1) * NP, :] for r in r2s], axis=0)
        gm_p = jnp.concatenate(
            [r[mid * NP:(mid + 1) * NP, :] for r in r2p], axis=0)
        acc = acc + dot(gm_s, wps_ref[mid]) + dot(gm_p, wpp_ref[mid])
    out_ref[...] = jnp.maximum(acc + lb_ref[...], 0.0) + df_ref[...]


def _tc_call(gf, lx2d, df, wpe1, bpe1, wpe2, bpe2,
             wn1, bn1, wn2, bn2, wn3, bn3, wps, wpp, lb):
    const2 = lambda i: (0, 0)
    row = lambda i: (i, 0)
    return pl.pallas_call(
        _tc_body,
        grid=(SGRID,),
        in_specs=[
            pl.BlockSpec((TROWS, C_IN // 2), row),
            pl.BlockSpec((TROWS, 128), row),
            pl.BlockSpec((TM, C_OUT), row),
            pl.BlockSpec((8, 64), const2),
            pl.BlockSpec((1, 64), const2),
            pl.BlockSpec((64, 32), const2),
            pl.BlockSpec((1, 32), const2),
            pl.BlockSpec((8, 8), const2),
            pl.BlockSpec((1, 8), const2),
            pl.BlockSpec((8, 8), const2),
            pl.BlockSpec((1, 8), const2),
            pl.BlockSpec((8, 16), const2),
            pl.BlockSpec((1, 16), const2),
            pl.BlockSpec((C_MID, C_IN, C_OUT), lambda i: (0, 0, 0)),
            pl.BlockSpec((C_MID, C_PE, C_OUT), lambda i: (0, 0, 0)),
            pl.BlockSpec((1, C_OUT), const2),
        ],
        out_specs=pl.BlockSpec((TM, C_OUT), row),
        out_shape=jax.ShapeDtypeStruct((SM, C_OUT), jnp.float32),
    )(gf, lx2d, df, wpe1, bpe1, wpe2, bpe2,
      wn1, bn1, wn2, bn2, wn3, bn3, wps, wpp, lb)


def kernel(sparse_xyz, sparse_feats, nei_inds, sparse_xyz_norm, dense_xyz,
           dense_xyz_norm, dense_feats, pe_W1, pe_b1, pe_W2, pe_b2,
           wn_W1, wn_b1, wn_W2, wn_b2, wn_W3, wn_b3, lin_W, lin_b):
    B, M, Kk = nei_inds.shape
    f32 = jnp.float32

    fbits = jax.lax.bitcast_convert_type(
        sparse_feats[0].astype(jnp.bfloat16), jnp.uint16)
    lo = fbits[:, :C_IN // 2].astype(jnp.uint32)
    hi = fbits[:, C_IN // 2:].astype(jnp.uint32) << 16
    feat_tbl = jax.lax.bitcast_convert_type(lo | hi, jnp.int32)
    xyz_tbl = jnp.pad(sparse_xyz[0].astype(f32), ((0, 0), (0, 125)))
    dxyz_flat = jnp.pad(dense_xyz[0].astype(f32),
                        ((0, M_PAD - M), (0, 13))).reshape(-1)
    idx = nei_inds[0].astype(jnp.int32).reshape(-1)
    idx = jnp.pad(idx, (0, ROWS - idx.shape[0]))

    df = jnp.pad(dense_feats[0].astype(f32), ((0, M_PAD - M), (0, 0)))

    wp = lin_W.reshape(C_IN + C_PE, C_MID, C_OUT).transpose(1, 0, 2)
    wps = wp[:, :C_IN, :]
    wpp = wp[:, C_IN:, :]
    wpe1 = jnp.pad(pe_W1, ((0, 5), (0, 0)))
    wn1 = jnp.pad(wn_W1, ((0, 5), (0, 0)))

    outs = []
    lxs = []
    for s in range(_NSLICE):
        idx_s = lax.slice(idx, (s * SROWS,), ((s + 1) * SROWS,))
        dxyz_s = lax.slice(dxyz_flat, (s * SM * 16,), ((s + 1) * SM * 16,))
        df_s = lax.slice(df, (s * SM, 0), ((s + 1) * SM, C_OUT))
        gfeat_s, lx2d_s = _sc_gather(feat_tbl, xyz_tbl, dxyz_s, idx_s)
        lxs.append(lx2d_s)
        outs.append(_tc_call(
            gfeat_s, lx2d_s, df_s,
            wpe1, pe_b1.reshape(1, -1), pe_W2, pe_b2.reshape(1, -1),
            wn1, wn_b1.reshape(1, -1), wn_W2, wn_b2.reshape(1, -1),
            wn_W3, wn_b3.reshape(1, -1), wps, wpp, lin_b.reshape(1, -1)))

    out1 = jnp.concatenate(outs, axis=0)
    lx2d = jnp.concatenate(lxs, axis=0)
    new_feat = out1[:M][None]
    localized = lx2d[:M * K, :3].reshape(1, M, K, 3)
    return new_feat, localized


# speedup vs baseline: 1.0971x; 1.0971x over previous
import jax
import jax.numpy as jnp
from jax import lax
from jax.experimental import pallas as pl
from jax.experimental.pallas import tpu as pltpu
from jax.experimental.pallas import tpu_sc as plsc

M_PAD = 10240
K = 16
N_SP = 2500
N_PAD = 2560
C_IN = 256
C_PE = 32
C_MID = 16
C_OUT = 256
ROWS = M_PAD * K
TM = 256
TROWS = TM * K
GRID = M_PAD // TM

_NSLICE = 4
SM = M_PAD // _NSLICE
SROWS = SM * K
SGRID = SM // TM

_NW = 32
_PER_W = SROWS // _NW
_PTS_W = SM // _NW
_CH = 128
_NSUP = _PER_W // _CH


def _sc_body(feat_tbl, xyz_tbl, dxyz_flat, idx_hbm,
             out_feat, out_lx,
             idx_all, f_a, f_b, xg_a, xg_b, dxt,
             sf_a, sf_b, sg_a, sg_b, sw_a, sw_b, sx_a, sx_b):
    f32 = jnp.float32
    wid = lax.axis_index("s") * 2 + lax.axis_index("c")
    rbase = pl.multiple_of(wid * _PER_W, _PER_W)

    pltpu.sync_copy(idx_hbm.at[pl.ds(rbase, _PER_W)], idx_all)
    pltpu.sync_copy(dxyz_flat.at[pl.ds(pl.multiple_of(wid * _PTS_W * 16,
                                                      _PTS_W * 16),
                                       _PTS_W * 16)], dxt)

    def issue_gathers(su, fbuf, xbuf, fsem, xsem):
        sl = idx_all.at[pl.ds(su * _CH, _CH)]
        hf = pltpu.async_copy(feat_tbl.at[sl], fbuf, fsem)
        hx = pltpu.async_copy(xyz_tbl.at[sl], xbuf, xsem)
        return hf, hx

    def lx_compute(su, xbuf):
        def rows8(r2, carry):
            for u in range(8):
                r = r2 * 8 + u
                p = su * 8 + (r >> 4)
                v = xbuf[r, pl.ds(0, 16)]
                d = dxt[pl.ds(p * 16, 16)]
                xbuf[r, pl.ds(0, 16)] = v - d
            return carry

        lax.fori_loop(0, _CH // 8, rows8, 0)

    def super_out(su, fbuf, xbuf, fsem, xsem):
        off = pl.multiple_of(rbase + su * _CH, _CH)
        wf = pltpu.async_copy(fbuf, out_feat.at[pl.ds(off, _CH)], fsem)
        wx = pltpu.async_copy(xbuf, out_lx.at[pl.ds(off, _CH)], xsem)
        return wf, wx

    def body(j, carry):
        su0 = 2 * j
        su1 = 2 * j + 1
        gf_a, gx_a = issue_gathers(su0, f_a, xg_a, sf_a, sg_a)
        gf_b, gx_b = issue_gathers(su1, f_b, xg_b, sf_b, sg_b)

        gf_a.wait()
        gx_a.wait()
        wf_a = pltpu.async_copy(
            f_a, out_feat.at[pl.ds(pl.multiple_of(rbase + su0 * _CH, _CH), _CH)],
            sw_a)
        lx_compute(su0, xg_a)
        wx_a = pltpu.async_copy(
            xg_a, out_lx.at[pl.ds(pl.multiple_of(rbase + su0 * _CH, _CH), _CH)],
            sx_a)

        gf_b.wait()
        gx_b.wait()
        wf_b = pltpu.async_copy(
            f_b, out_feat.at[pl.ds(pl.multiple_of(rbase + su1 * _CH, _CH), _CH)],
            sw_b)
        lx_compute(su1, xg_b)
        wx_b = pltpu.async_copy(
            xg_b, out_lx.at[pl.ds(pl.multiple_of(rbase + su1 * _CH, _CH), _CH)],
            sx_b)

        wf_a.wait()
        wx_a.wait()
        wf_b.wait()
        wx_b.wait()
        return carry

    lax.fori_loop(0, _NSUP // 2, body, 0)


def _sc_gather(feat_tbl, xyz_tbl, dxyz_flat, idx):
    f32 = jnp.float32
    sc_call = pl.kernel(
        _sc_body,
        out_type=[
            jax.ShapeDtypeStruct((SROWS, C_IN // 2), jnp.int32),
            jax.ShapeDtypeStruct((SROWS, 128), f32),
        ],
        mesh=plsc.VectorSubcoreMesh(core_axis_name="c", subcore_axis_name="s",
                                    num_cores=2),
        scratch_types=[
            pltpu.VMEM((_PER_W,), jnp.int32),
            pltpu.VMEM((_CH, C_IN // 2), jnp.int32),
            pltpu.VMEM((_CH, C_IN // 2), jnp.int32),
            pltpu.VMEM((_CH, 128), f32),
            pltpu.VMEM((_CH, 128), f32),
            pltpu.VMEM((_PTS_W * 16,), f32),
            pltpu.SemaphoreType.DMA,
            pltpu.SemaphoreType.DMA,
            pltpu.SemaphoreType.DMA,
            pltpu.SemaphoreType.DMA,
            pltpu.SemaphoreType.DMA,
            pltpu.SemaphoreType.DMA,
            pltpu.SemaphoreType.DMA,
            pltpu.SemaphoreType.DMA,
        ],
    )
    return sc_call(feat_tbl, xyz_tbl, dxyz_flat, idx)


def _tc_body(gf_ref, lx_ref,  df_ref,
             wpe1_ref, bpe1_ref, wpe2_ref, bpe2_ref,
             wn1_ref, bn1_ref, wn2_ref, bn2_ref, wn3_ref, bn3_ref,
             wps_ref, wpp_ref, lb_ref,
             out_ref):
    f32 = jnp.float32
    lx8 = lx_ref[...][:, :8]

    def dot(a, b):
        return jnp.dot(a, b, preferred_element_type=f32)

    h = jnp.maximum(dot(lx8, wpe1_ref[...]) + bpe1_ref[...], 0.0)
    fpe = jnp.maximum(dot(h, wpe2_ref[...]) + bpe2_ref[...], 0.0)
    w1 = jnp.maximum(dot(lx8, wn1_ref[...]) + bn1_ref[...], 0.0)
    w2 = jnp.maximum(dot(w1, wn2_ref[...]) + bn2_ref[...], 0.0)
    wgt = jnp.maximum(dot(w2, wn3_ref[...]) + bn3_ref[...], 0.0)

    wgtT = wgt.T
    NP = 16
    NG = TM // NP
    GR = NP * K
    col = lax.broadcasted_iota(jnp.int32, (NP, GR), 1)
    rowp = lax.broadcasted_iota(jnp.int32, (NP, GR), 0)
    maskp = (col // K == rowp).astype(f32)

    x = gf_ref[...]
    gf_lo = lax.bitcast_convert_type(x << 16, f32)
    gf_hi = lax.bitcast_convert_type(x & jnp.int32(-65536), f32)
    gf2 = jnp.concatenate([gf_lo, gf_hi], axis=1)
    r2s = []
    r2p = []
    for g in range(NG):
        o3 = (wgtT[:, None, g * GR:(g + 1) * GR] * maskp[None, :, :])
        o3r = o3.reshape(GR, GR)
        r2s.append(dot(o3r, gf2[g * GR:(g + 1) * GR, :]))
        r2p.append(dot(o3r, fpe[g * GR:(g + 1) * GR, :]))

    acc = jnp.zeros((TM, C_OUT), f32)
    for mid in range(C_MID):
        gm_s = jnp.concatenate(
            [r[mid * NP:(mid + 1) * NP, :] for r in r2s], axis=0)
        gm_p = jnp.concatenate(
            [r[mid * NP:(mid + 1) * NP, :] for r in r2p], axis=0)
        acc = acc + dot(gm_s, wps_ref[mid]) + dot(gm_p, wpp_ref[mid])
    out_ref[...] = jnp.maximum(acc + lb_ref[...], 0.0) + df_ref[...]


def _tc_call(gf, lx2d, df, wpe1, bpe1, wpe2, bpe2,
             wn1, bn1, wn2, bn2, wn3, bn3, wps, wpp, lb):
    const2 = lambda i: (0, 0)
    row = lambda i: (i, 0)
    return pl.pallas_call(
        _tc_body,
        grid=(SGRID,),
        in_specs=[
            pl.BlockSpec((TROWS, C_IN // 2), row),
            pl.BlockSpec((TROWS, 128), row),
            pl.BlockSpec((TM, C_OUT), row),
            pl.BlockSpec((8, 64), const2),
            pl.BlockSpec((1, 64), const2),
            pl.BlockSpec((64, 32), const2),
            pl.BlockSpec((1, 32), const2),
            pl.BlockSpec((8, 8), const2),
            pl.BlockSpec((1, 8), const2),
            pl.BlockSpec((8, 8), const2),
            pl.BlockSpec((1, 8), const2),
            pl.BlockSpec((8, 16), const2),
            pl.BlockSpec((1, 16), const2),
            pl.BlockSpec((C_MID, C_IN, C_OUT), lambda i: (0, 0, 0)),
            pl.BlockSpec((C_MID, C_PE, C_OUT), lambda i: (0, 0, 0)),
            pl.BlockSpec((1, C_OUT), const2),
        ],
        out_specs=pl.BlockSpec((TM, C_OUT), row),
        out_shape=jax.ShapeDtypeStruct((SM, C_OUT), jnp.float32),
    )(gf, lx2d, df, wpe1, bpe1, wpe2, bpe2,
      wn1, bn1, wn2, bn2, wn3, bn3, wps, wpp, lb)


def kernel(sparse_xyz, sparse_feats, nei_inds, sparse_xyz_norm, dense_xyz,
           dense_xyz_norm, dense_feats, pe_W1, pe_b1, pe_W2, pe_b2,
           wn_W1, wn_b1, wn_W2, wn_b2, wn_W3, wn_b3, lin_W, lin_b):
    B, M, Kk = nei_inds.shape
    f32 = jnp.float32

    fbits = jax.lax.bitcast_convert_type(
        sparse_feats[0].astype(jnp.bfloat16), jnp.uint16)
    lo = fbits[:, :C_IN // 2].astype(jnp.uint32)
    hi = fbits[:, C_IN // 2:].astype(jnp.uint32) << 16
    feat_tbl = jax.lax.bitcast_convert_type(lo | hi, jnp.int32)
    xyz_tbl = jnp.pad(sparse_xyz[0].astype(f32), ((0, 0), (0, 125)))
    dxyz_flat = jnp.pad(dense_xyz[0].astype(f32),
                        ((0, M_PAD - M), (0, 13))).reshape(-1)
    idx = nei_inds[0].astype(jnp.int32).reshape(-1)
    idx = jnp.pad(idx, (0, ROWS - idx.shape[0]))

    df = jnp.pad(dense_feats[0].astype(f32), ((0, M_PAD - M), (0, 0)))

    wp = lin_W.reshape(C_IN + C_PE, C_MID, C_OUT).transpose(1, 0, 2)
    wps = wp[:, :C_IN, :]
    wpp = wp[:, C_IN:, :]
    wpe1 = jnp.pad(pe_W1, ((0, 5), (0, 0)))
    wn1 = jnp.pad(wn_W1, ((0, 5), (0, 0)))

    outs = []
    lxs = []
    for s in range(_NSLICE):
        idx_s = lax.slice(idx, (s * SROWS,), ((s + 1) * SROWS,))
        dxyz_s = lax.slice(dxyz_flat, (s * SM * 16,), ((s + 1) * SM * 16,))
        df_s = lax.slice(df, (s * SM, 0), ((s + 1) * SM, C_OUT))
        if s >= 2:
            idx_s, _ = lax.optimization_barrier((idx_s, outs[s - 2]))
        gfeat_s, lx2d_s = _sc_gather(feat_tbl, xyz_tbl, dxyz_s, idx_s)
        lxs.append(lx2d_s)
        outs.append(_tc_call(
            gfeat_s, lx2d_s, df_s,
            wpe1, pe_b1.reshape(1, -1), pe_W2, pe_b2.reshape(1, -1),
            wn1, wn_b1.reshape(1, -1), wn_W2, wn_b2.reshape(1, -1),
            wn_W3, wn_b3.reshape(1, -1), wps, wpp, lin_b.reshape(1, -1)))

    out1 = jnp.concatenate(outs, axis=0)
    lx2d = jnp.concatenate(lxs, axis=0)
    new_feat = out1[:M][None]
    localized = lx2d[:M * K, :3].reshape(1, M, K, 3)
    return new_feat, localized


# speedup vs baseline: 1.2415x; 1.1316x over previous
import jax
import jax.numpy as jnp
from jax import lax
from jax.experimental import pallas as pl
from jax.experimental.pallas import tpu as pltpu
from jax.experimental.pallas import tpu_sc as plsc

M_PAD = 10240
K = 16
N_SP = 2500
N_PAD = 2560
C_IN = 256
C_PE = 32
C_MID = 16
C_OUT = 256
ROWS = M_PAD * K
TM = 256
TROWS = TM * K
GRID = M_PAD // TM

_NSLICE = 4
SM = M_PAD // _NSLICE
SROWS = SM * K
SGRID = SM // TM

_NW = 32
_PER_W = SROWS // _NW
_PTS_W = SM // _NW
_CH = 128
_NSUP = _PER_W // _CH


def _sc_body(feat_tbl, xyz_tbl, dxyz_flat, idx_hbm,
             out_feat, out_lx,
             idx_all, f_a, f_b, xg_a, xg_b, dxt,
             sf_a, sf_b, sg_a, sg_b, sw_a, sw_b, sx_a, sx_b):
    f32 = jnp.float32
    wid = lax.axis_index("s") * 2 + lax.axis_index("c")
    rbase = pl.multiple_of(wid * _PER_W, _PER_W)

    pltpu.sync_copy(idx_hbm.at[pl.ds(rbase, _PER_W)], idx_all)
    pltpu.sync_copy(dxyz_flat.at[pl.ds(pl.multiple_of(wid * _PTS_W * 16,
                                                      _PTS_W * 16),
                                       _PTS_W * 16)], dxt)

    def issue_gathers(su, fbuf, xbuf, fsem, xsem):
        sl = idx_all.at[pl.ds(su * _CH, _CH)]
        hf = pltpu.async_copy(feat_tbl.at[sl], fbuf, fsem)
        hx = pltpu.async_copy(xyz_tbl.at[sl], xbuf, xsem)
        return hf, hx

    def lx_compute(su, xbuf):
        def rows8(r2, carry):
            for u in range(8):
                r = r2 * 8 + u
                p = su * 8 + (r >> 4)
                v = xbuf[r, pl.ds(0, 16)]
                d = dxt[pl.ds(p * 16, 16)]
                xbuf[r, pl.ds(0, 16)] = v - d
            return carry

        lax.fori_loop(0, _CH // 8, rows8, 0)

    def super_out(su, fbuf, xbuf, fsem, xsem):
        off = pl.multiple_of(rbase + su * _CH, _CH)
        wf = pltpu.async_copy(fbuf, out_feat.at[pl.ds(off, _CH)], fsem)
        wx = pltpu.async_copy(xbuf, out_lx.at[pl.ds(off, _CH)], xsem)
        return wf, wx

    def body(j, carry):
        su0 = 2 * j
        su1 = 2 * j + 1
        gf_a, gx_a = issue_gathers(su0, f_a, xg_a, sf_a, sg_a)
        gf_b, gx_b = issue_gathers(su1, f_b, xg_b, sf_b, sg_b)

        gf_a.wait()
        gx_a.wait()
        wf_a = pltpu.async_copy(
            f_a, out_feat.at[pl.ds(pl.multiple_of(rbase + su0 * _CH, _CH), _CH)],
            sw_a)
        lx_compute(su0, xg_a)
        wx_a = pltpu.async_copy(
            xg_a, out_lx.at[pl.ds(pl.multiple_of(rbase + su0 * _CH, _CH), _CH)],
            sx_a)

        gf_b.wait()
        gx_b.wait()
        wf_b = pltpu.async_copy(
            f_b, out_feat.at[pl.ds(pl.multiple_of(rbase + su1 * _CH, _CH), _CH)],
            sw_b)
        lx_compute(su1, xg_b)
        wx_b = pltpu.async_copy(
            xg_b, out_lx.at[pl.ds(pl.multiple_of(rbase + su1 * _CH, _CH), _CH)],
            sx_b)

        wf_a.wait()
        wx_a.wait()
        wf_b.wait()
        wx_b.wait()
        return carry

    lax.fori_loop(0, _NSUP // 2, body, 0)


def _sc_gather(feat_tbl, xyz_tbl, dxyz_flat, idx):
    f32 = jnp.float32
    sc_call = pl.kernel(
        _sc_body,
        out_type=[
            jax.ShapeDtypeStruct((SROWS, C_IN // 2), jnp.int32),
            jax.ShapeDtypeStruct((SROWS, 128), f32),
        ],
        mesh=plsc.VectorSubcoreMesh(core_axis_name="c", subcore_axis_name="s",
                                    num_cores=2),
        scratch_types=[
            pltpu.VMEM((_PER_W,), jnp.int32),
            pltpu.VMEM((_CH, C_IN // 2), jnp.int32),
            pltpu.VMEM((_CH, C_IN // 2), jnp.int32),
            pltpu.VMEM((_CH, 128), f32),
            pltpu.VMEM((_CH, 128), f32),
            pltpu.VMEM((_PTS_W * 16,), f32),
            pltpu.SemaphoreType.DMA,
            pltpu.SemaphoreType.DMA,
            pltpu.SemaphoreType.DMA,
            pltpu.SemaphoreType.DMA,
            pltpu.SemaphoreType.DMA,
            pltpu.SemaphoreType.DMA,
            pltpu.SemaphoreType.DMA,
            pltpu.SemaphoreType.DMA,
        ],
    )
    return sc_call(feat_tbl, xyz_tbl, dxyz_flat, idx)


def _tc_body(gf_ref, lx_ref,  df_ref,
             wpe1_ref, bpe1_ref, wpe2_ref, bpe2_ref,
             wn1_ref, bn1_ref, wn2_ref, bn2_ref, wn3_ref, bn3_ref,
             wps_ref, wpp_ref, lb_ref,
             out_ref, lxo_ref):
    f32 = jnp.float32
    lx8 = lx_ref[...][:, :8]
    lxo_ref[...] = lx8

    def dot(a, b):
        return jnp.dot(a, b, preferred_element_type=f32)

    h = jnp.maximum(dot(lx8, wpe1_ref[...]) + bpe1_ref[...], 0.0)
    fpe = jnp.maximum(dot(h, wpe2_ref[...]) + bpe2_ref[...], 0.0)
    w1 = jnp.maximum(dot(lx8, wn1_ref[...]) + bn1_ref[...], 0.0)
    w2 = jnp.maximum(dot(w1, wn2_ref[...]) + bn2_ref[...], 0.0)
    wgt = jnp.maximum(dot(w2, wn3_ref[...]) + bn3_ref[...], 0.0)

    wgtT = wgt.T
    NP = 16
    NG = TM // NP
    GR = NP * K
    col = lax.broadcasted_iota(jnp.int32, (NP, GR), 1)
    rowp = lax.broadcasted_iota(jnp.int32, (NP, GR), 0)
    maskp = (col // K == rowp).astype(f32)

    x = gf_ref[...]
    gf_lo = lax.bitcast_convert_type(x << 16, f32)
    gf_hi = lax.bitcast_convert_type(x & jnp.int32(-65536), f32)
    gf2 = jnp.concatenate([gf_lo, gf_hi], axis=1)
    r2s = []
    r2p = []
    for g in range(NG):
        o3 = (wgtT[:, None, g * GR:(g + 1) * GR] * maskp[None, :, :])
        o3r = o3.reshape(GR, GR)
        r2s.append(dot(o3r, gf2[g * GR:(g + 1) * GR, :]))
        r2p.append(dot(o3r, fpe[g * GR:(g + 1) * GR, :]))

    acc = jnp.zeros((TM, C_OUT), f32)
    for mid in range(C_MID):
        gm_s = jnp.concatenate(
            [r[mid * NP:(mid + 1) * NP, :] for r in r2s], axis=0)
        gm_p = jnp.concatenate(
            [r[mid * NP:(mid + 1) * NP, :] for r in r2p], axis=0)
        acc = acc + dot(gm_s, wps_ref[mid]) + dot(gm_p, wpp_ref[mid])
    out_ref[...] = jnp.maximum(acc + lb_ref[...], 0.0) + df_ref[...]


def _tc_call(gf, lx2d, df, wpe1, bpe1, wpe2, bpe2,
             wn1, bn1, wn2, bn2, wn3, bn3, wps, wpp, lb):
    const2 = lambda i: (0, 0)
    row = lambda i: (i, 0)
    return pl.pallas_call(
        _tc_body,
        grid=(SGRID,),
        in_specs=[
            pl.BlockSpec((TROWS, C_IN // 2), row),
            pl.BlockSpec((TROWS, 128), row),
            pl.BlockSpec((TM, C_OUT), row),
            pl.BlockSpec((8, 64), const2),
            pl.BlockSpec((1, 64), const2),
            pl.BlockSpec((64, 32), const2),
            pl.BlockSpec((1, 32), const2),
            pl.BlockSpec((8, 8), const2),
            pl.BlockSpec((1, 8), const2),
            pl.BlockSpec((8, 8), const2),
            pl.BlockSpec((1, 8), const2),
            pl.BlockSpec((8, 16), const2),
            pl.BlockSpec((1, 16), const2),
            pl.BlockSpec((C_MID, C_IN, C_OUT), lambda i: (0, 0, 0)),
            pl.BlockSpec((C_MID, C_PE, C_OUT), lambda i: (0, 0, 0)),
            pl.BlockSpec((1, C_OUT), const2),
        ],
        out_specs=[pl.BlockSpec((TM, C_OUT), row),
                   pl.BlockSpec((TROWS, 8), row)],
        out_shape=[jax.ShapeDtypeStruct((SM, C_OUT), jnp.float32),
                   jax.ShapeDtypeStruct((SROWS, 8), jnp.float32)],
    )(gf, lx2d, df, wpe1, bpe1, wpe2, bpe2,
      wn1, bn1, wn2, bn2, wn3, bn3, wps, wpp, lb)


def kernel(sparse_xyz, sparse_feats, nei_inds, sparse_xyz_norm, dense_xyz,
           dense_xyz_norm, dense_feats, pe_W1, pe_b1, pe_W2, pe_b2,
           wn_W1, wn_b1, wn_W2, wn_b2, wn_W3, wn_b3, lin_W, lin_b):
    B, M, Kk = nei_inds.shape
    f32 = jnp.float32

    fbits = jax.lax.bitcast_convert_type(
        sparse_feats[0].astype(jnp.bfloat16), jnp.uint16)
    lo = fbits[:, :C_IN // 2].astype(jnp.uint32)
    hi = fbits[:, C_IN // 2:].astype(jnp.uint32) << 16
    feat_tbl = jax.lax.bitcast_convert_type(lo | hi, jnp.int32)
    xyz_tbl = jnp.pad(sparse_xyz[0].astype(f32), ((0, 0), (0, 125)))
    dxyz_flat = jnp.pad(dense_xyz[0].astype(f32),
                        ((0, M_PAD - M), (0, 13))).reshape(-1)
    idx = nei_inds[0].astype(jnp.int32).reshape(-1)
    idx = jnp.pad(idx, (0, ROWS - idx.shape[0]))

    df = jnp.pad(dense_feats[0].astype(f32), ((0, M_PAD - M), (0, 0)))

    wp = lin_W.reshape(C_IN + C_PE, C_MID, C_OUT).transpose(1, 0, 2)
    wps = wp[:, :C_IN, :]
    wpp = wp[:, C_IN:, :]
    wpe1 = jnp.pad(pe_W1, ((0, 5), (0, 0)))
    wn1 = jnp.pad(wn_W1, ((0, 5), (0, 0)))

    outs = []
    lxs = []
    for s in range(_NSLICE):
        idx_s = lax.slice(idx, (s * SROWS,), ((s + 1) * SROWS,))
        dxyz_s = lax.slice(dxyz_flat, (s * SM * 16,), ((s + 1) * SM * 16,))
        df_s = lax.slice(df, (s * SM, 0), ((s + 1) * SM, C_OUT))
        if s >= 2:
            idx_s, _ = lax.optimization_barrier((idx_s, outs[s - 2]))
        gfeat_s, lx2d_s = _sc_gather(feat_tbl, xyz_tbl, dxyz_s, idx_s)
        o_s, lxo_s = _tc_call(
            gfeat_s, lx2d_s, df_s,
            wpe1, pe_b1.reshape(1, -1), pe_W2, pe_b2.reshape(1, -1),
            wn1, wn_b1.reshape(1, -1), wn_W2, wn_b2.reshape(1, -1),
            wn_W3, wn_b3.reshape(1, -1), wps, wpp, lin_b.reshape(1, -1))
        outs.append(o_s)
        lxs.append(lxo_s)

    out1 = jnp.concatenate(outs, axis=0)
    new_feat = out1[:M][None]
    lxs[0], _ = lax.optimization_barrier((lxs[0], out1))
    lx2d = jnp.concatenate(lxs, axis=0)
    localized = lx2d[:M * K, :3].reshape(1, M, K, 3)
    return new_feat, localized
